# baseline (speedup 1.0000x reference)
import jax
import jax.numpy as jnp
import numpy as np
from jax.experimental import pallas as pl

N = 10000
E = 320000
IN = 128
OUT = 128
H = 8
C = OUT // H


def _ln_body(y_ref, x_ref, g_ref, b_ref, o_ref):
    y = y_ref[...] + x_ref[...]
    mean = jnp.mean(y, axis=-1, keepdims=True)
    var = jnp.mean((y - mean) ** 2, axis=-1, keepdims=True)
    o_ref[...] = (y - mean) * jax.lax.rsqrt(var + 1e-5) * g_ref[...] + b_ref[...]


def kernel(x, edge_index, W, att_src, att_dst, bias, gamma, beta):
    src = edge_index[0]
    dst = edge_index[1]
    loop = jnp.arange(N, dtype=src.dtype)
    src = jnp.concatenate([src, loop])
    dst = jnp.concatenate([dst, loop])

    xh = (x @ W).reshape(N, H, C)
    a_src = jnp.sum(xh * att_src[None, :, :], axis=-1)
    a_dst = jnp.sum(xh * att_dst[None, :, :], axis=-1)

    alpha = a_src[src] + a_dst[dst]
    alpha = jax.nn.leaky_relu(alpha, negative_slope=0.2)

    amax = jax.ops.segment_max(alpha, dst, num_segments=N)
    amax = jnp.where(jnp.isfinite(amax), amax, 0.0)
    ex = jnp.exp(alpha - amax[dst])
    denom = jax.ops.segment_sum(ex, dst, num_segments=N)
    att = ex / jnp.maximum(denom[dst], 1e-16)

    msg = xh[src] * att[..., None]
    out = jax.ops.segment_sum(msg, dst, num_segments=N)
    out = out.reshape(N, H * C) + bias

    blk = 400
    g2 = gamma.reshape(1, OUT)
    b2 = beta.reshape(1, OUT)
    return pl.pallas_call(
        _ln_body,
        grid=(N // blk,),
        in_specs=[
            pl.BlockSpec((blk, OUT), lambda i: (i, 0)),
            pl.BlockSpec((blk, OUT), lambda i: (i, 0)),
            pl.BlockSpec((1, OUT), lambda i: (0, 0)),
            pl.BlockSpec((1, OUT), lambda i: (0, 0)),
        ],
        out_specs=pl.BlockSpec((blk, OUT), lambda i: (i, 0)),
        out_shape=jax.ShapeDtypeStruct((N, OUT), jnp.float32),
    )(out, x, g2, b2)


# speedup vs baseline: 51.7613x; 51.7613x over previous
import functools

import jax
import jax.numpy as jnp
from jax import lax
from jax.experimental import pallas as pl
from jax.experimental.pallas import tpu as pltpu
from jax.experimental.pallas import tpu_sc as plsc

N = 10000
E = 320000
IN = 128
OUT = 128
H = 8
C = OUT // H
NE = E + N
NC, NS, L = 2, 16, 16
NW = NC * NS
G = 128
K = 81
T = G * K
EP = T * NW
NEG = 0.2

_MESH = dict(core_axis_name="c", subcore_axis_name="s", num_cores=NC,
             num_subcores=NS)


def _mm_body(x_ref, w_ref, a_ref, xh_ref, ad_ref):
    xh = jnp.dot(x_ref[...], w_ref[...], preferred_element_type=jnp.float32)
    xh_ref[...] = xh
    ad_ref[...] = jnp.dot(xh, a_ref[...], preferred_element_type=jnp.float32)


def _stage_a(x, W, A):
    blk = 400
    return pl.pallas_call(
        _mm_body,
        grid=(N // blk,),
        in_specs=[
            pl.BlockSpec((blk, IN), lambda i: (i, 0)),
            pl.BlockSpec((IN, OUT), lambda i: (0, 0)),
            pl.BlockSpec((OUT, 2 * H), lambda i: (0, 0)),
        ],
        out_specs=[
            pl.BlockSpec((blk, OUT), lambda i: (i, 0)),
            pl.BlockSpec((blk, 2 * H), lambda i: (i, 0)),
        ],
        out_shape=[
            jax.ShapeDtypeStruct((N, OUT), jnp.float32),
            jax.ShapeDtypeStruct((N, 2 * H), jnp.float32),
        ],
    )(x, W, A)


@functools.partial(
    pl.kernel,
    out_type=(jax.ShapeDtypeStruct((EP, H), jnp.float32),
              jax.ShapeDtypeStruct((NC, N, H), jnp.float32)),
    mesh=plsc.VectorSubcoreMesh(**_MESH),
    compiler_params=pltpu.CompilerParams(needs_layout_passes=False, use_tc_tiling_on_sc=False),
    scratch_types=[
        pltpu.VMEM((G,), jnp.int32),
        pltpu.VMEM((G,), jnp.int32),
        pltpu.VMEM((G, 2 * H), jnp.float32),
        pltpu.VMEM((G, 2 * H), jnp.float32),
        pltpu.VMEM((G, H), jnp.float32),
        pltpu.VMEM_SHARED((N, H), jnp.float32),
        pltpu.SemaphoreType.DMA,
        pltpu.SemaphoreType.DMA,
    ],
)
def _pass1(ad_hbm, srcp_hbm, dstp_hbm, zero8_hbm, ex_hbm, dpart_hbm,
           src_v, dst_v, ads_v, add_v, ex_v, den_sp, sem1, sem2):
    c = lax.axis_index("c")
    s = lax.axis_index("s")
    wid = c * NS + s

    @pl.when(s == 0)
    def _init():
        pltpu.sync_copy(zero8_hbm, den_sp)

    plsc.subcore_barrier()
    iota = lax.iota(jnp.int32, L)

    def chunk(k, carry):
        base = (wid * K + k) * G
        pltpu.sync_copy(srcp_hbm.at[pl.ds(base, G)], src_v)
        pltpu.sync_copy(dstp_hbm.at[pl.ds(base, G)], dst_v)
        cp1 = pltpu.async_copy(ad_hbm.at[src_v], ads_v, sem1)
        cp2 = pltpu.async_copy(ad_hbm.at[dst_v], add_v, sem2)
        cp1.wait()
        cp2.wait()

        def group(g, carry2):
            e_loc = g * L + iota
            mask = (base + e_loc) < NE
            for h in range(H):
                hs = jnp.full((L,), h, jnp.int32)
                hd = jnp.full((L,), H + h, jnp.int32)
                a = (plsc.load_gather(ads_v, [e_loc, hs])
                     + plsc.load_gather(add_v, [e_loc, hd]))
                a = jnp.where(a < 0, a * NEG, a)
                exv = jnp.where(mask, jnp.exp(a), 0.0)
                plsc.store_scatter(ex_v, [e_loc, hs], exv)
            return carry2

        lax.fori_loop(0, G // L, group, 0)
        pltpu.sync_copy(ex_v, den_sp.at[dst_v], add=True)
        pltpu.sync_copy(ex_v, ex_hbm.at[pl.ds(base, G)])
        return carry

    lax.fori_loop(0, K, chunk, 0)
    plsc.subcore_barrier()

    @pl.when(s == 0)
    def _fin():
        pltpu.sync_copy(den_sp, dpart_hbm.at[c])


def _rd_body(dp_ref, rd_ref):
    r = 1.0 / (dp_ref[0] + dp_ref[1])
    rd_ref[...] = jnp.concatenate([r, r], axis=-1)


def _stage_c(dparts):
    blk = 400
    return pl.pallas_call(
        _rd_body,
        grid=(N // blk,),
        in_specs=[pl.BlockSpec((NC, blk, H), lambda i: (0, i, 0))],
        out_specs=pl.BlockSpec((blk, 2 * H), lambda i: (i, 0)),
        out_shape=jax.ShapeDtypeStruct((N, 2 * H), jnp.float32),
    )(dparts)


def _bcast(v, j):
    idx = jnp.full((L, 1), j, jnp.int32)
    dn = lax.GatherDimensionNumbers(offset_dims=(), collapsed_slice_dims=(0,),
                                    start_index_map=(0,))
    return lax.gather(v, idx, dn, (1,),
                      mode=lax.GatherScatterMode.PROMISE_IN_BOUNDS)


@functools.partial(
    pl.kernel,
    out_type=jax.ShapeDtypeStruct((NC, N, OUT), jnp.float32),
    mesh=plsc.VectorSubcoreMesh(**_MESH),
    compiler_params=pltpu.CompilerParams(needs_layout_passes=False, use_tc_tiling_on_sc=False),
    scratch_types=[
        pltpu.VMEM((G,), jnp.int32),
        pltpu.VMEM((G,), jnp.int32),
        pltpu.VMEM((G, OUT), jnp.float32),
        pltpu.VMEM((G, H), jnp.float32),
        pltpu.VMEM((G, 2 * H), jnp.float32),
        pltpu.VMEM_SHARED((N, OUT), jnp.float32),
        pltpu.SemaphoreType.DMA,
        pltpu.SemaphoreType.DMA,
    ],
)
def _pass2(xh_hbm, srcp_hbm, dstp_hbm, ex_hbm, rd_hbm, zeroO_hbm, opart_hbm,
           src_v, dst_v, rows_v, ex_v, rd_v, out_sp, sem1, sem2):
    c = lax.axis_index("c")
    s = lax.axis_index("s")
    wid = c * NS + s

    @pl.when(s == 0)
    def _init():
        pltpu.sync_copy(zeroO_hbm, out_sp)

    plsc.subcore_barrier()
    iota = lax.iota(jnp.int32, L)
    mask8 = iota < H

    def chunk(k, carry):
        base = (wid * K + k) * G
        pltpu.sync_copy(srcp_hbm.at[pl.ds(base, G)], src_v)
        pltpu.sync_copy(dstp_hbm.at[pl.ds(base, G)], dst_v)
        cp1 = pltpu.async_copy(xh_hbm.at[src_v], rows_v, sem1)
        cp2 = pltpu.async_copy(rd_hbm.at[dst_v], rd_v, sem2)
        pltpu.sync_copy(ex_hbm.at[pl.ds(base, G)], ex_v)
        cp1.wait()
        cp2.wait()

        def edge(e, carry2):
            ef = jnp.full((L,), e, jnp.int32)
            ex_row = plsc.load_gather(ex_v, [ef, iota], mask=mask8)
            rd_row = plsc.load_gather(rd_v, [ef, iota])
            att = ex_row * rd_row
            for j in range(H):
                bj = _bcast(att, j)
                r = rows_v[e, pl.ds(j * L, L)]
                rows_v[e, pl.ds(j * L, L)] = r * bj
            return carry2

        lax.fori_loop(0, G, edge, 0)
        pltpu.sync_copy(rows_v, out_sp.at[dst_v], add=True)
        return carry

    lax.fori_loop(0, K, chunk, 0)
    plsc.subcore_barrier()

    @pl.when(s == 0)
    def _fin():
        pltpu.sync_copy(out_sp, opart_hbm.at[c])


def _fin_body(op_ref, x_ref, b_ref, g_ref, be_ref, o_ref):
    y = op_ref[0] + op_ref[1] + b_ref[...] + x_ref[...]
    mean = jnp.mean(y, axis=-1, keepdims=True)
    var = jnp.mean((y - mean) ** 2, axis=-1, keepdims=True)
    o_ref[...] = (y - mean) * lax.rsqrt(var + 1e-5) * g_ref[...] + be_ref[...]


def _stage_e(oparts, x, bias, gamma, beta):
    blk = 400
    return pl.pallas_call(
        _fin_body,
        grid=(N // blk,),
        in_specs=[
            pl.BlockSpec((NC, blk, OUT), lambda i: (0, i, 0)),
            pl.BlockSpec((blk, OUT), lambda i: (i, 0)),
            pl.BlockSpec((1, OUT), lambda i: (0, 0)),
            pl.BlockSpec((1, OUT), lambda i: (0, 0)),
            pl.BlockSpec((1, OUT), lambda i: (0, 0)),
        ],
        out_specs=pl.BlockSpec((blk, OUT), lambda i: (i, 0)),
        out_shape=jax.ShapeDtypeStruct((N, OUT), jnp.float32),
    )(oparts, x, bias, gamma, beta)


def kernel(x, edge_index, W, att_src, att_dst, bias, gamma, beta):
    src = edge_index[0]
    dst = edge_index[1]
    loop = jnp.arange(N, dtype=jnp.int32)
    pad = jnp.zeros((EP - NE,), jnp.int32)
    srcp = jnp.concatenate([src, loop, pad])
    dstp = jnp.concatenate([dst, loop, pad])

    eyeH = jnp.eye(H, dtype=jnp.float32)
    A1 = (att_src[:, :, None] * eyeH[:, None, :]).reshape(OUT, H)
    A2 = (att_dst[:, :, None] * eyeH[:, None, :]).reshape(OUT, H)
    A = jnp.concatenate([A1, A2], axis=1)

    xh, ad = _stage_a(x, W, A)
    zero8 = jnp.zeros((N, H), jnp.float32)
    ex, dparts = _pass1(ad, srcp, dstp, zero8)
    rd = _stage_c(dparts)
    zeroO = jnp.zeros((N, OUT), jnp.float32)
    oparts = _pass2(xh, srcp, dstp, ex, rd, zeroO)
    return _stage_e(oparts, x, bias.reshape(1, OUT), gamma.reshape(1, OUT),
                    beta.reshape(1, OUT))
